# Initial kernel scaffold; baseline (speedup 1.0000x reference)
import jax
import jax.numpy as jnp
from jax.experimental import pallas as pl


def kernel(x_sc, x_fc, pool_w, multi_w, multi_b):
    raise NotImplementedError("write your pallas kernel here")


# speedup vs baseline: 1.7061x; 1.7061x over previous
import functools

import jax
import jax.numpy as jnp
from jax import lax
from jax.experimental import pallas as pl
from jax.experimental.pallas import tpu as pltpu
from jax.experimental.pallas import tpu_sc as plsc

_N = 90
_D = 512
_K = 45
_B = 1024

_GA = 32
_RA = _GA * _N


def _score_body(xs_ref, xf_ref, pw_ref, mw_ref, mb_ref, out_ref):
    w = pw_ref[...]
    wn = jnp.sqrt(jnp.sum(w * w))
    ssc = jnp.tanh(jnp.dot(xs_ref[...], w, preferred_element_type=jnp.float32) / wn)
    sfc = jnp.tanh(jnp.dot(xf_ref[...], w, preferred_element_type=jnp.float32) / wn)
    m = mw_ref[0, 0] * ssc + mw_ref[0, 1] * sfc + mb_ref[...]
    out_ref[...] = jax.nn.sigmoid(m)


def _scores(x_sc, x_fc, pool_w, multi_w, multi_b):
    mb_tiled = jnp.tile(multi_b, _GA).reshape(_RA, 1)
    mw2 = multi_w.reshape(1, 2)
    grid = (_B * _N) // _RA
    return pl.pallas_call(
        _score_body,
        grid=(grid,),
        in_specs=[
            pl.BlockSpec((_RA, _D), lambda i: (i, 0)),
            pl.BlockSpec((_RA, _D), lambda i: (i, 0)),
            pl.BlockSpec((_D, 1), lambda i: (0, 0)),
            pl.BlockSpec((1, 2), lambda i: (0, 0)),
            pl.BlockSpec((_RA, 1), lambda i: (0, 0)),
        ],
        out_specs=pl.BlockSpec((_RA, 1), lambda i: (i, 0)),
        out_shape=jax.ShapeDtypeStruct((_B * _N, 1), jnp.float32),
    )(x_sc, x_fc, pool_w, mw2, mb_tiled)


_GB = 8


def _topk_body(ms_ref, ids_ref, ss_ref):
    s = ms_ref[...]
    si = s[:, :, None]
    sj = s[:, None, :]
    ii = lax.broadcasted_iota(jnp.int32, (_GB, _N, _N), 1)
    jj = lax.broadcasted_iota(jnp.int32, (_GB, _N, _N), 2)
    before = (sj > si) | ((sj == si) & (jj < ii))
    rank = jnp.sum(before.astype(jnp.int32), axis=2)
    rr = lax.broadcasted_iota(jnp.int32, (_GB, _N, _N), 1)
    oh = (rank[:, None, :] == rr)
    idx_i = lax.broadcasted_iota(jnp.int32, (_GB, _N, _N), 2)
    inv = jnp.sum(jnp.where(oh, idx_i, 0), axis=2)
    sv = jnp.sum(jnp.where(oh, s[:, None, :], 0.0), axis=2)
    g0 = pl.program_id(0) * _GB
    gid = g0 + lax.broadcasted_iota(jnp.int32, (_GB, _N), 0)
    ids_ref[...] = (gid * _N + inv)[:, :_K]
    ss_ref[...] = sv[:, :_K]


def _topk(ms):
    grid = _B // _GB
    return pl.pallas_call(
        _topk_body,
        grid=(grid,),
        in_specs=[pl.BlockSpec((_GB, _N), lambda i: (i, 0))],
        out_specs=[
            pl.BlockSpec((_GB, _K), lambda i: (i, 0)),
            pl.BlockSpec((_GB, _K), lambda i: (i, 0)),
        ],
        out_shape=[
            jax.ShapeDtypeStruct((_B, _K), jnp.int32),
            jax.ShapeDtypeStruct((_B, _K), jnp.float32),
        ],
    )(ms)


_NW = 32
_RW = (_B * _K) // _NW
_CH = 96
_NCH = _RW // _CH
_CSL = _D // 16


def _gather_scale_body(xs_hbm, xf_hbm, ids_hbm, ss_hbm, osc_hbm, ofc_hbm,
                       idx_v, s_v, rsc_v, rfc_v, sem1, sem2):
    nc = 2
    wid = lax.axis_index("s") * nc + lax.axis_index("c")
    base = wid * _RW
    pltpu.sync_copy(ids_hbm.at[pl.ds(base, _RW)], idx_v)
    pltpu.sync_copy(ss_hbm.at[pl.ds(base, _RW)], s_v)

    def scale_rows(rows_ref, s_off):
        def grp_body(g, _):
            sv16 = s_v[pl.ds(s_off + g * 16, 16)]
            for k in range(16):
                splat = jnp.full((16,), sv16[k])
                r = g * 16 + k
                for c in range(_CSL):
                    sl = pl.ds(c * 16, 16)
                    rows_ref[r, sl] = rows_ref[r, sl] * splat
            return 0
        lax.fori_loop(0, _CH // 16, grp_body, 0)

    def chunk_body(ch, _):
        off = ch * _CH
        idx = idx_v.at[pl.ds(off, _CH)]
        cp1 = pltpu.make_async_copy(xs_hbm.at[idx], rsc_v, sem1)
        cp2 = pltpu.make_async_copy(xf_hbm.at[idx], rfc_v, sem2)
        cp1.start()
        cp2.start()
        cp1.wait()
        scale_rows(rsc_v, off)
        cp2.wait()
        scale_rows(rfc_v, off)
        pltpu.sync_copy(rsc_v, osc_hbm.at[pl.ds(base + off, _CH)])
        pltpu.sync_copy(rfc_v, ofc_hbm.at[pl.ds(base + off, _CH)])
        return 0

    lax.fori_loop(0, _NCH, chunk_body, 0)


def _gather_scale(x_sc, x_fc, row_ids, s_sorted):
    mesh = plsc.VectorSubcoreMesh(core_axis_name="c", subcore_axis_name="s")
    f = pl.kernel(
        _gather_scale_body,
        out_type=[
            jax.ShapeDtypeStruct((_B * _K, _D), jnp.float32),
            jax.ShapeDtypeStruct((_B * _K, _D), jnp.float32),
        ],
        mesh=mesh,
        scratch_types=[
            pltpu.VMEM((_RW,), jnp.int32),
            pltpu.VMEM((_RW,), jnp.float32),
            pltpu.VMEM((_CH, _D), jnp.float32),
            pltpu.VMEM((_CH, _D), jnp.float32),
            pltpu.SemaphoreType.DMA,
            pltpu.SemaphoreType.DMA,
        ],
    )
    return f(x_sc, x_fc, row_ids, s_sorted)


def kernel(x_sc, x_fc, pool_w, multi_w, multi_b):
    ms = _scores(x_sc, x_fc, pool_w, multi_w, multi_b)
    row_ids, s_sorted = _topk(ms.reshape(_B, _N))
    out_sc, out_fc = _gather_scale(
        x_sc, x_fc, row_ids.reshape(-1), s_sorted.reshape(-1))
    return (out_sc.reshape(_B, _K, _D), out_fc.reshape(_B, _K, _D))
